# baseline (speedup 1.0000x reference)
import functools

import jax
import jax.numpy as jnp
from jax import lax
from jax.experimental import pallas as pl
from jax.experimental.pallas import tpu as pltpu
from jax.experimental.pallas import tpu_sc as plsc

BATCH = 16384
EMBED_K = 32
NUM_WORKERS = 32
B_PER_W = BATCH // NUM_WORKERS


def _sc_gather(user_table, item_table, user_idx, item_idx):
    mesh = plsc.VectorSubcoreMesh(core_axis_name="c", subcore_axis_name="s")
    row_t = jax.ShapeDtypeStruct((BATCH, EMBED_K), jnp.float32)

    @functools.partial(
        pl.kernel,
        mesh=mesh,
        out_type=[row_t, row_t],
        compiler_params=pltpu.CompilerParams(use_tc_tiling_on_sc=False),
        scratch_types=[
            pltpu.VMEM((B_PER_W,), jnp.int32),
            pltpu.VMEM((B_PER_W, EMBED_K), jnp.float32),
            pltpu.VMEM((B_PER_W,), jnp.int32),
            pltpu.VMEM((B_PER_W, EMBED_K), jnp.float32),
            pltpu.SemaphoreType.DMA,
            pltpu.SemaphoreType.DMA,
        ],
    )
    def gather_kernel(u_tab, i_tab, u_idx, i_idx, u_out, i_out,
                      u_idx_v, u_rows_v, i_idx_v, i_rows_v, sem_u, sem_i):
        wid = lax.axis_index("s") * 2 + lax.axis_index("c")
        base = wid * B_PER_W
        pltpu.sync_copy(u_idx.at[pl.ds(base, B_PER_W)], u_idx_v)
        pltpu.sync_copy(i_idx.at[pl.ds(base, B_PER_W)], i_idx_v)
        cu = pltpu.async_copy(u_tab.at[u_idx_v], u_rows_v, sem_u)
        ci = pltpu.async_copy(i_tab.at[i_idx_v], i_rows_v, sem_i)
        cu.wait()
        ci.wait()
        pltpu.sync_copy(u_rows_v, u_out.at[pl.ds(base, B_PER_W)])
        pltpu.sync_copy(i_rows_v, i_out.at[pl.ds(base, B_PER_W)])

    return gather_kernel(user_table, item_table, user_idx, item_idx)


def _tc_body(u_ref, i_ref, out_ref, uc_ref, ic_ref):
    u = u_ref[...]
    v = i_ref[...]
    u = u - jnp.mean(u, axis=1, keepdims=True)
    v = v - jnp.mean(v, axis=1, keepdims=True)
    uc_ref[...] = u
    ic_ref[...] = v
    out_ref[...] = jnp.sum(u * v, axis=1, keepdims=True)


def _tc_center_dot(u_raw, i_raw):
    blk = 2048
    grid = BATCH // blk
    emb_spec = pl.BlockSpec((blk, EMBED_K), lambda b: (b, 0))
    return pl.pallas_call(
        _tc_body,
        grid=(grid,),
        in_specs=[emb_spec, emb_spec],
        out_specs=[pl.BlockSpec((blk, 1), lambda b: (b, 0)), emb_spec, emb_spec],
        out_shape=[
            jax.ShapeDtypeStruct((BATCH, 1), jnp.float32),
            jax.ShapeDtypeStruct((BATCH, EMBED_K), jnp.float32),
            jax.ShapeDtypeStruct((BATCH, EMBED_K), jnp.float32),
        ],
    )(u_raw, i_raw)


def kernel(x, user_table, item_table, scale_param):
    user_idx = x[:, 0]
    item_idx = x[:, 1]
    u_raw, i_raw = _sc_gather(user_table, item_table, user_idx, item_idx)
    out, u_c, i_c = _tc_center_dot(u_raw, i_raw)
    return (out, u_c, i_c)


# speedup vs baseline: 1.0077x; 1.0077x over previous
import functools

import jax
import jax.numpy as jnp
from jax import lax
from jax.experimental import pallas as pl
from jax.experimental.pallas import tpu as pltpu
from jax.experimental.pallas import tpu_sc as plsc

BATCH = 16384
EMBED_K = 32
NUM_WORKERS = 32
B_PER_W = BATCH // NUM_WORKERS
LANES = 16
PACK = 128 // EMBED_K
N_LINES = 1000000 // PACK


def _sc_fused(p_u, p_i, user_idx, item_idx):
    mesh = plsc.VectorSubcoreMesh(core_axis_name="c", subcore_axis_name="s")
    emb_t = jax.ShapeDtypeStruct((EMBED_K, BATCH), jnp.float32)
    dot_t = jax.ShapeDtypeStruct((BATCH,), jnp.float32)

    @functools.partial(
        pl.kernel,
        mesh=mesh,
        out_type=[dot_t, emb_t, emb_t],
        compiler_params=pltpu.CompilerParams(needs_layout_passes=False),
        scratch_types=[
            pltpu.VMEM((B_PER_W,), jnp.int32),
            pltpu.VMEM((B_PER_W,), jnp.int32),
            pltpu.VMEM((B_PER_W,), jnp.int32),
            pltpu.VMEM((B_PER_W, 128), jnp.float32),
            pltpu.VMEM((EMBED_K, B_PER_W), jnp.float32),
            pltpu.VMEM((EMBED_K, B_PER_W), jnp.float32),
            pltpu.VMEM((B_PER_W,), jnp.float32),
            pltpu.SemaphoreType.DMA,
        ],
    )
    def fused_kernel(u_tab, i_tab, u_idx, i_idx, dot_out, uc_out, ic_out,
                     idx_v, g_v, idx2_v, lines_v, s_u, s_i, dot_v, sem):
        wid = lax.axis_index("s") * 2 + lax.axis_index("c")
        base = wid * B_PER_W

        iota = lax.iota(jnp.int32, LANES)

        def gather_extract(tab, idx_hbm, s_out):
            pltpu.sync_copy(idx_hbm.at[pl.ds(base, B_PER_W)], idx_v)

            @pl.loop(0, B_PER_W, step=LANES)
            def _(j):
                sl = pl.ds(j, LANES)
                g_v[sl] = lax.shift_right_logical(idx_v[sl], 2)
            pltpu.async_copy(tab.at[g_v], lines_v, sem).wait()

            @pl.loop(0, B_PER_W, step=LANES)
            def _(j):
                sl = pl.ds(j, LANES)
                r_vec = idx_v[sl]
                o_vec = (r_vec & 3) * EMBED_K
                row_vec = iota + j
                for c in range(EMBED_K):
                    s_out[c, sl] = plsc.load_gather(
                        lines_v, [row_vec, o_vec + c])

        gather_extract(u_tab, u_idx, s_u)
        gather_extract(i_tab, i_idx, s_i)

        @pl.loop(0, B_PER_W, step=LANES)
        def _(j):
            sl = pl.ds(j, LANES)
            u_sum = s_u[0, sl]
            i_sum = s_i[0, sl]
            for c in range(1, EMBED_K):
                u_sum = u_sum + s_u[c, sl]
                i_sum = i_sum + s_i[c, sl]
            u_mean = u_sum * (1.0 / EMBED_K)
            i_mean = i_sum * (1.0 / EMBED_K)
            acc = jnp.zeros((LANES,), jnp.float32)
            for c in range(EMBED_K):
                u_cent = s_u[c, sl] - u_mean
                i_cent = s_i[c, sl] - i_mean
                s_u[c, sl] = u_cent
                s_i[c, sl] = i_cent
                acc = acc + u_cent * i_cent
            dot_v[sl] = acc

        pltpu.sync_copy(s_u, uc_out.at[:, pl.ds(base, B_PER_W)])
        pltpu.sync_copy(s_i, ic_out.at[:, pl.ds(base, B_PER_W)])
        pltpu.sync_copy(dot_v, dot_out.at[pl.ds(base, B_PER_W)])

    return fused_kernel(p_u, p_i, user_idx, item_idx)


def kernel(x, user_table, item_table, scale_param):
    user_idx = x[:, 0]
    item_idx = x[:, 1]
    p_u = user_table.reshape(N_LINES, 128)
    p_i = item_table.reshape(N_LINES, 128)
    dot, uc_t, ic_t = _sc_fused(p_u, p_i, user_idx, item_idx)
    return (dot[:, None], uc_t.T, ic_t.T)
